# baseline (speedup 1.0000x reference)
import jax
import jax.numpy as jnp
from jax import lax
from jax.experimental import pallas as pl
from jax.experimental.pallas import tpu as pltpu

_B = 32
_T = 27
_N = 27
_NN = _N * _N
_DE = 5
_DH = 32
_DW = 20
_NWK = 2000
_DDEC = 42

_KF = _N * _DE
_HF = _N * _DH
_TG = 27

_F32 = jnp.float32


def _edge_body(e_ref, wb_ref, o_ref):
    for i in range(_TG):
        o_ref[0, 0, i] = jnp.dot(e_ref[0, 0, i], wb_ref[...],
                                 preferred_element_type=_F32)


def _main_body(sidx_ref, widx_ref, e_ref, wb_ref, em_ref, eedsq_ref, eedf_ref,
               esdsq_ref, esdf_ref, vt_ref, s_ref, vpt_ref, vdt_ref, vnum_ref,
               dm_ref, wtab_ref, wn_ref, ws_ref, bs_ref,
               edge_o, eed_o, esd_o, nodeh_o, vval_o, vdy_o, dec_o, wt_o):
    for t in range(_T):
        edge_o[0, t] = jnp.dot(e_ref[0, t], wb_ref[...],
                               preferred_element_type=_F32)
    sidx = sidx_ref[0]
    oh = (lax.broadcasted_iota(jnp.int32, (_T, _N), 1) == sidx).astype(_F32)
    eedg = jnp.dot(oh, eedsq_ref[0], preferred_element_type=_F32)
    esdg = jnp.dot(oh, esdsq_ref[0], preferred_element_type=_F32)
    sf = jnp.dot(oh, s_ref[0], preferred_element_type=_F32)
    t_c = sf[:, 3:4]

    dec_o[0] = jnp.dot(sf, ws_ref[...],
                       preferred_element_type=_F32) + bs_ref[...]

    dm = dm_ref[0]
    ch3 = vpt_ref[0] - t_c
    ch4 = t_c - vdt_ref[0]
    ch5 = eedg * dm
    ch6 = esdg * dm

    vdy_o[0, 0] = ch5
    vdy_o[0, 1] = ch6

    vval_o[0, 0] = vt_ref[0, 0:1, :] * dm
    vval_o[0, 1] = vt_ref[0, 1:2, :] * dm
    vval_o[0, 2] = vt_ref[0, 2:3, :] * dm
    vval_o[0, 3] = ch3 * dm
    vval_o[0, 4] = ch4 * dm
    vval_o[0, 5] = ch5 * dm
    vval_o[0, 6] = ch6 * dm
    vval_o[0, 7] = vnum_ref[0] * dm

    for t in range(_T):
        vv_t = vval_o[0, :, t, :]
        nodeh_o[0, t] = lax.dot_general(
            vv_t, wn_ref[...], (((0,), (0,)), ((), ())),
            preferred_element_type=_F32)

    em = em_ref[0]
    eed_o[0] = eedf_ref[0] * em
    esd_o[0] = esdf_ref[0] * em

    ohw = (lax.broadcasted_iota(jnp.int32, (1, _NWK), 1)
           == widx_ref[0]).astype(_F32)
    wt_o[0] = jnp.dot(ohw, wtab_ref[...], preferred_element_type=_F32)


def kernel(V, V_reach_mask, V_ft, V_pt, V_dt, V_num, V_dispatch_mask, E, E_ed,
           E_sd, E_mask, start_idx, cou, worker_table, W_node, W_edge, W_start,
           b_start):
    B, T, N = V_reach_mask.shape
    NN = N * N

    E_v = E.reshape(B, T, N, _KF)
    W_big = jnp.kron(jnp.eye(_N, dtype=_F32), W_edge)


    Em_r = E_mask.reshape(B, T, NN)
    eedf = E_ed.reshape(B, 1, NN)
    esdf = E_sd.reshape(B, 1, NN)
    V_T = V.transpose(0, 2, 1)
    S = jnp.concatenate([V, V_ft[..., None], V_dt[..., None]], axis=2)
    vpt = V_pt.reshape(B, 1, N)
    vdt = V_dt.reshape(B, 1, N)
    sidx = start_idx.astype(jnp.int32).reshape(B, T, 1)
    widx = cou[:, 0].astype(jnp.int32).reshape(B, 1, 1)
    bs = b_start.reshape(1, _DDEC)

    full = lambda shp: pl.BlockSpec(shp, lambda b: (0,) * len(shp))
    per_b = lambda shp: pl.BlockSpec(shp, lambda b: (b,) + (0,) * (len(shp) - 1))

    in_specs = [
        per_b((1, T, 1)),
        per_b((1, 1, 1)),
        per_b((1, T, N, _KF)),
        full((_KF, _HF)),
        per_b((1, T, NN)),
        per_b((1, N, N)),
        per_b((1, 1, NN)),
        per_b((1, N, N)),
        per_b((1, 1, NN)),
        per_b((1, 3, N)),
        per_b((1, N, _DE)),
        per_b((1, 1, N)),
        per_b((1, 1, N)),
        per_b((1, T, N)),
        per_b((1, T, N)),
        full((_NWK, _DW)),
        full((8, _DH)),
        full((_DE, _DDEC)),
        full((1, _DDEC)),
    ]
    out_specs = [
        per_b((1, T, N, _HF)),
        per_b((1, T, NN)),
        per_b((1, T, NN)),
        per_b((1, T, N, _DH)),
        per_b((1, 8, T, N)),
        per_b((1, 2, T, N)),
        per_b((1, T, _DDEC)),
        per_b((1, 1, _DW)),
    ]
    out_shape = [
        jax.ShapeDtypeStruct((B, T, N, _HF), _F32),
        jax.ShapeDtypeStruct((B, T, NN), _F32),
        jax.ShapeDtypeStruct((B, T, NN), _F32),
        jax.ShapeDtypeStruct((B, T, N, _DH), _F32),
        jax.ShapeDtypeStruct((B, 8, T, N), _F32),
        jax.ShapeDtypeStruct((B, 2, T, N), _F32),
        jax.ShapeDtypeStruct((B, T, _DDEC), _F32),
        jax.ShapeDtypeStruct((B, 1, _DW), _F32),
    ]

    outs = pl.pallas_call(
        _main_body,
        grid=(B,),
        in_specs=in_specs,
        out_specs=out_specs,
        out_shape=out_shape,
    )(sidx, widx, E_v, W_big, Em_r, E_ed, eedf, E_sd, esdf, V_T, S, vpt, vdt,
      V_num, V_dispatch_mask, worker_table, W_node, W_start, bs)

    edge_v, eed, esd, nodeh, vval, vdy, dec, wt = outs
    b_edge_h = edge_v.reshape(B, T, N, N, _DH)

    b_eed = eed.reshape(B, T, N, N)
    b_esd = esd.reshape(B, T, N, N)
    b_V_val = vval.transpose(0, 2, 3, 1)
    b_V_dy = vdy.transpose(0, 2, 3, 1)
    wt_g = wt.reshape(B, _DW)
    embed_cou = jnp.concatenate(
        [jnp.repeat(wt_g, T, axis=0), jnp.repeat(cou[:, 1:4], T, axis=0)],
        axis=1)

    return (nodeh, b_edge_h, dec, b_V_val, b_eed, b_esd, b_V_dy, embed_cou)


# speedup vs baseline: 1.0387x; 1.0387x over previous
import jax
import jax.numpy as jnp
from jax import lax
from jax.experimental import pallas as pl
from jax.experimental.pallas import tpu as pltpu

_B = 32
_T = 27
_N = 27
_NN = _N * _N
_DE = 5
_DH = 32
_DW = 20
_NWK = 2000
_DDEC = 42

_KF = _N * _DE
_HF = _N * _DH
_TG = 27

_F32 = jnp.float32


def _edge_body(e_ref, wb_ref, o_ref):
    for i in range(_TG):
        o_ref[0, 0, i] = jnp.dot(e_ref[0, 0, i], wb_ref[...],
                                 preferred_element_type=_F32)


def _main_body(sidx_ref, widx_ref, em_ref, eedsq_ref, eedf_ref, esdsq_ref,
               esdf_ref, vt_ref, s_ref, vpt_ref, vdt_ref, vnum_ref, dm_ref,
               wtab_ref, wn_ref, ws_ref, bs_ref,
               eed_o, esd_o, nodeh_o, vval_o, vdy_o, dec_o, wt_o):
    sidx = sidx_ref[0]
    oh = (lax.broadcasted_iota(jnp.int32, (_T, _N), 1) == sidx).astype(_F32)
    eedg = jnp.dot(oh, eedsq_ref[0], preferred_element_type=_F32)
    esdg = jnp.dot(oh, esdsq_ref[0], preferred_element_type=_F32)
    sf = jnp.dot(oh, s_ref[0], preferred_element_type=_F32)
    t_c = sf[:, 3:4]

    dec_o[0] = jnp.dot(sf, ws_ref[...],
                       preferred_element_type=_F32) + bs_ref[...]

    dm = dm_ref[0]
    ch3 = vpt_ref[0] - t_c
    ch4 = t_c - vdt_ref[0]
    ch5 = eedg * dm
    ch6 = esdg * dm

    vdy_o[0, 0] = ch5
    vdy_o[0, 1] = ch6

    vval_o[0, 0] = vt_ref[0, 0:1, :] * dm
    vval_o[0, 1] = vt_ref[0, 1:2, :] * dm
    vval_o[0, 2] = vt_ref[0, 2:3, :] * dm
    vval_o[0, 3] = ch3 * dm
    vval_o[0, 4] = ch4 * dm
    vval_o[0, 5] = ch5 * dm
    vval_o[0, 6] = ch6 * dm
    vval_o[0, 7] = vnum_ref[0] * dm

    for t in range(_T):
        vv_t = vval_o[0, :, t, :]
        nodeh_o[0, t] = lax.dot_general(
            vv_t, wn_ref[...], (((0,), (0,)), ((), ())),
            preferred_element_type=_F32)

    em = em_ref[0]
    eed_o[0] = eedf_ref[0] * em
    esd_o[0] = esdf_ref[0] * em

    ohw = (lax.broadcasted_iota(jnp.int32, (1, _NWK), 1)
           == widx_ref[0]).astype(_F32)
    wt_o[0] = jnp.dot(ohw, wtab_ref[...], preferred_element_type=_F32)


def kernel(V, V_reach_mask, V_ft, V_pt, V_dt, V_num, V_dispatch_mask, E, E_ed,
           E_sd, E_mask, start_idx, cou, worker_table, W_node, W_edge, W_start,
           b_start):
    B, T, N = V_reach_mask.shape
    NN = N * N

    E_v = E.reshape(B, T // _TG, _TG, N, _KF).astype(jnp.bfloat16)
    W_big = jnp.kron(jnp.eye(_N, dtype=_F32), W_edge).astype(jnp.bfloat16)

    edge_v = pl.pallas_call(
        _edge_body,
        grid=(B, T // _TG),
        in_specs=[
            pl.BlockSpec((1, 1, _TG, N, _KF), lambda b, g: (b, g, 0, 0, 0)),
            pl.BlockSpec((_KF, _HF), lambda b, g: (0, 0)),
        ],
        out_specs=pl.BlockSpec((1, 1, _TG, N, _HF), lambda b, g: (b, g, 0, 0, 0)),
        out_shape=jax.ShapeDtypeStruct((B, T // _TG, _TG, N, _HF), _F32),
    )(E_v, W_big)
    b_edge_h = edge_v.reshape(B, T, N, N, _DH)

    Em_r = E_mask.reshape(B, T, NN)
    eedf = E_ed.reshape(B, 1, NN)
    esdf = E_sd.reshape(B, 1, NN)
    V_T = V.transpose(0, 2, 1)
    S = jnp.concatenate([V, V_ft[..., None], V_dt[..., None]], axis=2)
    vpt = V_pt.reshape(B, 1, N)
    vdt = V_dt.reshape(B, 1, N)
    sidx = start_idx.astype(jnp.int32).reshape(B, T, 1)
    widx = cou[:, 0].astype(jnp.int32).reshape(B, 1, 1)
    bs = b_start.reshape(1, _DDEC)

    full = lambda shp: pl.BlockSpec(shp, lambda b: (0,) * len(shp))
    per_b = lambda shp: pl.BlockSpec(shp, lambda b: (b,) + (0,) * (len(shp) - 1))

    in_specs = [
        per_b((1, T, 1)),
        per_b((1, 1, 1)),
        per_b((1, T, NN)),
        per_b((1, N, N)),
        per_b((1, 1, NN)),
        per_b((1, N, N)),
        per_b((1, 1, NN)),
        per_b((1, 3, N)),
        per_b((1, N, _DE)),
        per_b((1, 1, N)),
        per_b((1, 1, N)),
        per_b((1, T, N)),
        per_b((1, T, N)),
        full((_NWK, _DW)),
        full((8, _DH)),
        full((_DE, _DDEC)),
        full((1, _DDEC)),
    ]
    out_specs = [
        per_b((1, T, NN)),
        per_b((1, T, NN)),
        per_b((1, T, N, _DH)),
        per_b((1, 8, T, N)),
        per_b((1, 2, T, N)),
        per_b((1, T, _DDEC)),
        per_b((1, 1, _DW)),
    ]
    out_shape = [
        jax.ShapeDtypeStruct((B, T, NN), _F32),
        jax.ShapeDtypeStruct((B, T, NN), _F32),
        jax.ShapeDtypeStruct((B, T, N, _DH), _F32),
        jax.ShapeDtypeStruct((B, 8, T, N), _F32),
        jax.ShapeDtypeStruct((B, 2, T, N), _F32),
        jax.ShapeDtypeStruct((B, T, _DDEC), _F32),
        jax.ShapeDtypeStruct((B, 1, _DW), _F32),
    ]

    outs = pl.pallas_call(
        _main_body,
        grid=(B,),
        in_specs=in_specs,
        out_specs=out_specs,
        out_shape=out_shape,
    )(sidx, widx, Em_r, E_ed, eedf, E_sd, esdf, V_T, S, vpt, vdt,
      V_num, V_dispatch_mask, worker_table, W_node, W_start, bs)

    eed, esd, nodeh, vval, vdy, dec, wt = outs

    b_eed = eed.reshape(B, T, N, N)
    b_esd = esd.reshape(B, T, N, N)
    b_V_val = vval.transpose(0, 2, 3, 1)
    b_V_dy = vdy.transpose(0, 2, 3, 1)
    wt_g = wt.reshape(B, _DW)
    embed_cou = jnp.concatenate(
        [jnp.repeat(wt_g, T, axis=0), jnp.repeat(cou[:, 1:4], T, axis=0)],
        axis=1)

    return (nodeh, b_edge_h, dec, b_V_val, b_eed, b_esd, b_V_dy, embed_cou)
